# baseline (speedup 1.0000x reference)
import functools

import jax
import jax.numpy as jnp
from jax.experimental import pallas as pl
from jax.experimental.pallas import tpu as pltpu

N = 4096
D = 128
BLK = 512


def _degree_body(thr_ref, a_ref, dout_ref, din_ref):
    i = pl.program_id(0)
    mask = (a_ref[...] >= thr_ref[0]).astype(jnp.float32)
    dout_ref[...] = jnp.sum(mask, axis=1)[None, :]

    @pl.when(i == 0)
    def _():
        din_ref[...] = jnp.zeros_like(din_ref)

    din_ref[...] += jnp.sum(mask, axis=0)[None, :]


def _degrees(A, thr):
    grid = (N // BLK,)
    dout, din = pl.pallas_call(
        _degree_body,
        grid=grid,
        in_specs=[
            pl.BlockSpec(memory_space=pltpu.SMEM),
            pl.BlockSpec((BLK, N), lambda i: (i, 0)),
        ],
        out_specs=[
            pl.BlockSpec((1, BLK), lambda i: (0, i)),
            pl.BlockSpec((1, N), lambda i: (0, 0)),
        ],
        out_shape=[
            jax.ShapeDtypeStruct((1, N), jnp.float32),
            jax.ShapeDtypeStruct((1, N), jnp.float32),
        ],
        compiler_params=pltpu.CompilerParams(
            dimension_semantics=("arbitrary",),
        ),
    )(thr, A)
    return dout[0], din[0]


def _layer_body(thr_ref, a_ref, h_src_ref, h_dst_ref, no_src_ref,
                no_dst_ref, ni_ref, w_ref, b_ref, out_ref, acc_ref):
    k = pl.program_id(1)
    nk = pl.num_programs(1)

    @pl.when(k == 0)
    def _():
        acc_ref[...] = jnp.zeros_like(acc_ref)

    mask = (a_ref[...] >= thr_ref[0]).astype(jnp.float32)
    hs = h_src_ref[...] * no_src_ref[...].T
    acc_ref[...] += jax.lax.dot_general(
        mask, hs, (((0,), (0,)), ((), ())),
        preferred_element_type=jnp.float32)

    @pl.when(k == nk - 1)
    def _():
        agg = acc_ref[...] + h_dst_ref[...] * no_dst_ref[...].T
        hd = agg * ni_ref[...].T
        out_ref[...] = jax.nn.relu(
            jnp.dot(hd, w_ref[...], preferred_element_type=jnp.float32)
            + b_ref[...])


def _layer(A, h, norm_out, norm_in, W, b, thr):
    grid = (N // BLK, N // BLK)
    no2 = norm_out[None, :]
    ni2 = norm_in[None, :]
    return pl.pallas_call(
        _layer_body,
        grid=grid,
        in_specs=[
            pl.BlockSpec(memory_space=pltpu.SMEM),
            pl.BlockSpec((BLK, BLK), lambda j, k: (k, j)),
            pl.BlockSpec((BLK, D), lambda j, k: (k, 0)),
            pl.BlockSpec((BLK, D), lambda j, k: (j, 0)),
            pl.BlockSpec((1, BLK), lambda j, k: (0, k)),
            pl.BlockSpec((1, BLK), lambda j, k: (0, j)),
            pl.BlockSpec((1, BLK), lambda j, k: (0, j)),
            pl.BlockSpec((D, D), lambda j, k: (0, 0)),
            pl.BlockSpec((1, D), lambda j, k: (0, 0)),
        ],
        out_specs=pl.BlockSpec((BLK, D), lambda j, k: (j, 0)),
        out_shape=jax.ShapeDtypeStruct((N, D), jnp.float32),
        scratch_shapes=[pltpu.VMEM((BLK, D), jnp.float32)],
        compiler_params=pltpu.CompilerParams(
            dimension_semantics=("parallel", "arbitrary"),
        ),
    )(thr, A, h, h, no2, no2, ni2, W, b[None, :])


def kernel(A, features, threshold, W1, b1, W2, b2):
    thr = jnp.asarray(threshold, jnp.float32).reshape(1)
    dout, din = _degrees(A, thr)
    norm_out = jax.lax.rsqrt(dout + 1.0)
    norm_in = jax.lax.rsqrt(din + 1.0)
    h1 = _layer(A, features, norm_out, norm_in, W1, b1, thr)
    h2 = _layer(A, h1, norm_out, norm_in, W2, b2, thr)
    return (h1, h2)
